# baseline (speedup 1.0000x reference)
import functools

import jax
import jax.numpy as jnp
import numpy as np
from jax import lax
from jax.experimental import pallas as pl
from jax.experimental.pallas import tpu as pltpu
from jax.experimental.pallas import tpu_sc as plsc

D = 10000
LEVELS = 1024
HALF = 5120
RINGW = 5136
NT = 68
NCHUNK = HALF // 16
SIGNM = np.int32(-(2**31))


def _sc_partials(e_flat, idx_flat, k_flat):
    mesh = plsc.VectorSubcoreMesh(core_axis_name="c", subcore_axis_name="s")

    @functools.partial(
        pl.kernel,
        mesh=mesh,
        out_type=jax.ShapeDtypeStruct((32 * HALF,), jnp.float32),
        scratch_types=[
            pltpu.VMEM((272,), jnp.int32),
            pltpu.VMEM((272,), jnp.int32),
            pltpu.VMEM((4, HALF), jnp.float32),
            pltpu.VMEM((8, HALF), jnp.float32),
            pltpu.VMEM((8, HALF), jnp.float32),
            pltpu.VMEM((3 * RINGW,), jnp.float32),
            pltpu.VMEM((HALF,), jnp.float32),
            pltpu.SemaphoreType.DMA,
            pltpu.SemaphoreType.DMA,
        ],
    )
    def k(e_hbm, idx_hbm, k_hbm, out_hbm, idx_v, idxa_v, key_v, buf_a, buf_b, ring_v, acc_v, sem_a, sem_b):
        c = lax.axis_index("c")
        s = lax.axis_index("s")
        b = s // 2
        tc = s % 2
        t0 = tc * 64

        pltpu.sync_copy(idx_hbm.at[pl.ds((b * 128 + t0) * 4, 272)], idx_v)
        pltpu.sync_copy(k_hbm.at[pl.ds(c * 4, 4)], key_v)

        row0 = c * LEVELS

        def adj_body(j, _):
            v = idx_v[pl.ds(j * 16, 16)]
            idxa_v[pl.ds(j * 16, 16)] = v + row0
            return 0

        lax.fori_loop(0, 17, adj_body, 0)

        zeros = jnp.zeros((16,), jnp.float32)

        def zero_body(j, _):
            o = j * 16
            acc_v[pl.ds(o, 16)] = zeros
            ring_v[pl.ds(o, 16)] = zeros
            ring_v[pl.ds(RINGW + o, 16)] = zeros
            ring_v[pl.ds(2 * RINGW + o, 16)] = zeros
            return 0

        lax.fori_loop(0, NCHUNK + 1, zero_body, 0)

        NP = NT // 2

        def issue(p, buf, sem):
            pltpu.async_copy(e_hbm.at[idxa_v.at[pl.ds(p * 8, 8)]], buf, sem)

        def drain(buf, sem):
            pltpu.make_async_copy(e_hbm.at[idxa_v.at[pl.ds(0, 8)]], buf, sem).wait()

        def compute_pair(g, buf_v):
            for sub in (0, 1):
                i = g * 2 + sub
                r0 = sub * 4
                slot = (i % 3) * RINGW
                rn = (i <= 65) & (t0 + i <= 127)

                def bind_chunk(j, r0=r0, slot=slot):
                    o = j * 16
                    sv = ((buf_v[r0 + 0, pl.ds(o, 16)] * key_v[0, pl.ds(o, 16)]
                           + buf_v[r0 + 1, pl.ds(o, 16)] * key_v[1, pl.ds(o, 16)])
                          + (buf_v[r0 + 2, pl.ds(o, 16)] * key_v[2, pl.ds(o, 16)]
                             + buf_v[r0 + 3, pl.ds(o, 16)] * key_v[3, pl.ds(o, 16)]))
                    ring_v[pl.ds(slot + o, 16)] = sv

                @pl.when(rn)
                def _():
                    @plsc.parallel_loop(0, NCHUNK, unroll=8)
                    def bind_body(j):
                        bind_chunk(j)

                @pl.when(rn & (i >= 2))
                def _():
                    s0 = ((i - 2) % 3) * RINGW
                    s1 = ((i - 1) % 3) * RINGW
                    s2 = slot

                    @plsc.parallel_loop(0, NCHUNK, unroll=8)
                    def win_body(j):
                        o = j * 16
                        a = ring_v[pl.ds(s0 + o, 16)]
                        bb = ring_v[pl.ds(s1 + o + 1, 16)]
                        cc = ring_v[pl.ds(s2 + o + 2, 16)]
                        plsc.addupdate(acc_v.at[pl.ds(o, 16)], a * bb * cc)

        issue(0, buf_a, sem_a)
        issue(1, buf_b, sem_b)

        def pair_body(h, _):
            drain(buf_a, sem_a)
            compute_pair(2 * h, buf_a)

            @pl.when(2 * h + 2 < NP)
            def _():
                issue(2 * h + 2, buf_a, sem_a)

            drain(buf_b, sem_b)
            compute_pair(2 * h + 1, buf_b)

            @pl.when(2 * h + 3 < NP)
            def _():
                issue(2 * h + 3, buf_b, sem_b)

            return 0

        lax.fori_loop(0, NP // 2, pair_body, 0)

        wid = c * 16 + s
        pltpu.sync_copy(acc_v, out_hbm.at[pl.ds(wid * HALF, HALF)])

    return k(e_flat, idx_flat, k_flat)


def kernel(signals, feat, keys_w, embed_w, mfcc_w, sin_w, sin_b):
    xs = jnp.clip(signals, 0.0, 1.0)
    idx = jnp.round(xs * (LEVELS - 1)).astype(jnp.int32)
    idx = jnp.clip(idx, 0, LEVELS - 1).reshape(1024, 4)
    idx = jnp.pad(idx, ((0, 8), (0, 0)))
    idx_flat = idx.reshape(-1)

    e_ext = jnp.concatenate([embed_w, embed_w[:, :HALF - 5000]], axis=1)
    e_flat = jnp.concatenate([e_ext[:, :HALF], e_ext[:, 5000:5000 + HALF]], axis=0)
    k_ext = jnp.concatenate([keys_w, keys_w[:, :HALF - 5000]], axis=1)
    k_flat = jnp.concatenate([k_ext[:, :HALF], k_ext[:, 5000:5000 + HALF]], axis=0)
    partials = _sc_partials(e_flat, idx_flat, k_flat)

    u = partials.reshape(2, 16, HALF).sum(axis=1)
    u = jnp.concatenate([u[0, :5000], u[1, :5000]])
    s_hv = jnp.roll(u, 2)

    fh = {}
    for si in (0, 3, 4, 5, 6, 11, 12, 15, 17):
        x = feat[546 + 3 * si: 549 + 3 * si]
        proj = sin_w[si] @ x
        fh[si] = jnp.cos(proj + sin_b[si]) * jnp.sin(proj)
    f = ((fh[0] + fh[15] + fh[17]) * (fh[3] + fh[4] + fh[5])
         * fh[6] * (fh[11] + fh[12]))

    return jnp.where((s_hv != 0) & (f > 0), 1.0, -1.0).astype(jnp.float32)


# speedup vs baseline: 1.1620x; 1.1620x over previous
import functools

import jax
import jax.numpy as jnp
import numpy as np
from jax import lax
from jax.experimental import pallas as pl
from jax.experimental.pallas import tpu as pltpu
from jax.experimental.pallas import tpu_sc as plsc

D = 10000
LEVELS = 1024
HALF = 5120
RINGW = 5136
NT = 68
NCHUNK = HALF // 16
SIGNM = np.int32(-(2**31))


def _sc_partials(e_tbl, halo_tbl, idx_flat, k_flat):
    mesh = plsc.VectorSubcoreMesh(core_axis_name="c", subcore_axis_name="s")

    @functools.partial(
        pl.kernel,
        mesh=mesh,
        out_type=jax.ShapeDtypeStruct((32 * HALF,), jnp.float32),
        scratch_types=[
            pltpu.VMEM((272,), jnp.int32),
            pltpu.VMEM((4, HALF), jnp.float32),
            pltpu.VMEM((8, HALF), jnp.float32),
            pltpu.VMEM((8, HALF), jnp.float32),
            pltpu.VMEM((3 * RINGW,), jnp.float32),
            pltpu.VMEM((HALF,), jnp.float32),
            pltpu.SemaphoreType.DMA,
            pltpu.SemaphoreType.DMA,
        ],
    )
    def k(e_hbm, h_hbm, idx_hbm, k_hbm, out_hbm, idx_v, key_v, buf_a, buf_b, ring_v, acc_v, sem_a, sem_b):
        c = lax.axis_index("c")
        s = lax.axis_index("s")
        b = s // 2
        tc = s % 2
        t0 = tc * 64

        pltpu.sync_copy(idx_hbm.at[pl.ds((b * 128 + t0) * 4, 272)], idx_v)
        pltpu.sync_copy(k_hbm.at[pl.ds(c * 4, 4)], key_v)


        zeros = jnp.zeros((16,), jnp.float32)

        def zero_body(j, _):
            o = j * 16
            acc_v[pl.ds(o, 16)] = zeros
            ring_v[pl.ds(o, 16)] = zeros
            ring_v[pl.ds(RINGW + o, 16)] = zeros
            ring_v[pl.ds(2 * RINGW + o, 16)] = zeros
            return 0

        lax.fori_loop(0, NCHUNK + 1, zero_body, 0)

        NP = NT // 2

        def issue(p, buf, sem):
            idx_sl = idx_v.at[pl.ds(p * 8, 8)]

            @pl.when(c == 0)
            def _():
                pltpu.async_copy(
                    e_hbm.at[idx_sl, pl.ds(0, HALF)], buf, sem)

            @pl.when(c == 1)
            def _():
                pltpu.async_copy(
                    e_hbm.at[idx_sl, pl.ds(4992, 4992)],
                    buf.at[:, pl.ds(0, 4992)], sem)
                pltpu.async_copy(
                    h_hbm.at[idx_sl], buf.at[:, pl.ds(4992, 128)], sem)

        def drain(buf, sem):
            idx_sl = idx_v.at[pl.ds(0, 8)]

            @pl.when(c == 0)
            def _():
                pltpu.make_async_copy(
                    e_hbm.at[idx_sl, pl.ds(0, HALF)], buf, sem).wait()

            @pl.when(c == 1)
            def _():
                pltpu.make_async_copy(
                    e_hbm.at[idx_sl, pl.ds(4992, 4992)],
                    buf.at[:, pl.ds(0, 4992)], sem).wait()
                pltpu.make_async_copy(
                    h_hbm.at[idx_sl], buf.at[:, pl.ds(4992, 128)], sem).wait()

        def compute_pair(g, buf_v):
            for sub in (0, 1):
                i = g * 2 + sub
                r0 = sub * 4
                slot = (i % 3) * RINGW
                rn = (i <= 65) & (t0 + i <= 127)

                def bind_chunk(j, r0=r0, slot=slot):
                    o = j * 16
                    sv = ((buf_v[r0 + 0, pl.ds(o, 16)] * key_v[0, pl.ds(o, 16)]
                           + buf_v[r0 + 1, pl.ds(o, 16)] * key_v[1, pl.ds(o, 16)])
                          + (buf_v[r0 + 2, pl.ds(o, 16)] * key_v[2, pl.ds(o, 16)]
                             + buf_v[r0 + 3, pl.ds(o, 16)] * key_v[3, pl.ds(o, 16)]))
                    ring_v[pl.ds(slot + o, 16)] = sv

                @pl.when(rn)
                def _():
                    @plsc.parallel_loop(0, NCHUNK, unroll=8)
                    def bind_body(j):
                        bind_chunk(j)

                @pl.when(rn & (i >= 2))
                def _():
                    s0 = ((i - 2) % 3) * RINGW
                    s1 = ((i - 1) % 3) * RINGW
                    s2 = slot

                    @plsc.parallel_loop(0, NCHUNK, unroll=8)
                    def win_body(j):
                        o = j * 16
                        a = ring_v[pl.ds(s0 + o, 16)]
                        bb = ring_v[pl.ds(s1 + o + 1, 16)]
                        cc = ring_v[pl.ds(s2 + o + 2, 16)]
                        plsc.addupdate(acc_v.at[pl.ds(o, 16)], a * bb * cc)

        issue(0, buf_a, sem_a)
        issue(1, buf_b, sem_b)

        def pair_body(h, _):
            drain(buf_a, sem_a)
            compute_pair(2 * h, buf_a)

            @pl.when(2 * h + 2 < NP)
            def _():
                issue(2 * h + 2, buf_a, sem_a)

            drain(buf_b, sem_b)
            compute_pair(2 * h + 1, buf_b)

            @pl.when(2 * h + 3 < NP)
            def _():
                issue(2 * h + 3, buf_b, sem_b)

            return 0

        lax.fori_loop(0, NP // 2, pair_body, 0)

        wid = c * 16 + s
        pltpu.sync_copy(acc_v, out_hbm.at[pl.ds(wid * HALF, HALF)])

    return k(e_tbl, halo_tbl, idx_flat, k_flat)


def kernel(signals, feat, keys_w, embed_w, mfcc_w, sin_w, sin_b):
    xs = jnp.clip(signals, 0.0, 1.0)
    idx = jnp.round(xs * (LEVELS - 1)).astype(jnp.int32)
    idx = jnp.clip(idx, 0, LEVELS - 1).reshape(1024, 4)
    idx = jnp.pad(idx, ((0, 8), (0, 0)))
    idx_flat = idx.reshape(-1)

    halo_tbl = jnp.concatenate([embed_w[:, 9984:], embed_w[:, :112]], axis=1)
    k_flat = jnp.concatenate(
        [keys_w[:, :HALF],
         jnp.concatenate([keys_w[:, 4992:], keys_w[:, :112]], axis=1)], axis=0)
    partials = _sc_partials(embed_w, halo_tbl, idx_flat, k_flat)

    u = partials.reshape(2, 16, HALF).sum(axis=1)
    u = jnp.concatenate([u[0, :5000], u[1, 8:5008]])
    s_hv = jnp.roll(u, 2)

    fh = {}
    for si in (0, 3, 4, 5, 6, 11, 12, 15, 17):
        x = feat[546 + 3 * si: 549 + 3 * si]
        proj = sin_w[si] @ x
        fh[si] = jnp.cos(proj + sin_b[si]) * jnp.sin(proj)
    f = ((fh[0] + fh[15] + fh[17]) * (fh[3] + fh[4] + fh[5])
         * fh[6] * (fh[11] + fh[12]))

    return jnp.where((s_hv != 0) & (f > 0), 1.0, -1.0).astype(jnp.float32)


# speedup vs baseline: 1.3615x; 1.1717x over previous
import functools

import jax
import jax.numpy as jnp
import numpy as np
from jax import lax
from jax.experimental import pallas as pl
from jax.experimental.pallas import tpu as pltpu
from jax.experimental.pallas import tpu_sc as plsc

D = 10000
LEVELS = 1024
HALF = 5120
RINGW = 5136
NT = 68
NCHUNK = HALF // 16
SIGNM = np.int32(-(2**31))


def _sc_partials(e_tbl, halo_tbl, idx_flat, k_flat):
    mesh = plsc.VectorSubcoreMesh(core_axis_name="c", subcore_axis_name="s")

    @functools.partial(
        pl.kernel,
        mesh=mesh,
        out_type=jax.ShapeDtypeStruct((32 * HALF,), jnp.float32),
        scratch_types=[
            pltpu.VMEM((272,), jnp.int32),
            pltpu.VMEM((4, HALF), jnp.float32),
            pltpu.VMEM((8, HALF), jnp.float32),
            pltpu.VMEM((8, HALF), jnp.float32),
            pltpu.VMEM((3 * RINGW,), jnp.float32),
            pltpu.VMEM((HALF,), jnp.float32),
            pltpu.SemaphoreType.DMA,
            pltpu.SemaphoreType.DMA,
        ],
    )
    def k(e_hbm, h_hbm, idx_hbm, k_hbm, out_hbm, idx_v, key_v, buf_a, buf_b, ring_v, acc_v, sem_a, sem_b):
        c = lax.axis_index("c")
        s = lax.axis_index("s")
        b = s // 2
        tc = s % 2
        t0 = tc * 64

        pltpu.sync_copy(idx_hbm.at[pl.ds((b * 128 + t0) * 4, 272)], idx_v)
        pltpu.sync_copy(k_hbm.at[pl.ds(c * 4, 4)], key_v)


        zeros = jnp.zeros((16,), jnp.float32)

        def zero_body(j, _):
            o = j * 16
            acc_v[pl.ds(o, 16)] = zeros
            ring_v[pl.ds(o, 16)] = zeros
            ring_v[pl.ds(RINGW + o, 16)] = zeros
            ring_v[pl.ds(2 * RINGW + o, 16)] = zeros
            return 0

        lax.fori_loop(0, NCHUNK + 1, zero_body, 0)

        NP = NT // 2

        def issue(p, buf, sem):
            idx_sl = idx_v.at[pl.ds(p * 8, 8)]

            @pl.when(c == 0)
            def _():
                pltpu.async_copy(
                    e_hbm.at[idx_sl, pl.ds(0, HALF)], buf, sem)

            @pl.when(c == 1)
            def _():
                pltpu.async_copy(
                    e_hbm.at[idx_sl, pl.ds(4992, 4992)],
                    buf.at[:, pl.ds(0, 4992)], sem)
                pltpu.async_copy(
                    h_hbm.at[idx_sl], buf.at[:, pl.ds(4992, 128)], sem)

        def drain(buf, sem):
            idx_sl = idx_v.at[pl.ds(0, 8)]

            @pl.when(c == 0)
            def _():
                pltpu.make_async_copy(
                    e_hbm.at[idx_sl, pl.ds(0, HALF)], buf, sem).wait()

            @pl.when(c == 1)
            def _():
                pltpu.make_async_copy(
                    e_hbm.at[idx_sl, pl.ds(4992, 4992)],
                    buf.at[:, pl.ds(0, 4992)], sem).wait()
                pltpu.make_async_copy(
                    h_hbm.at[idx_sl], buf.at[:, pl.ds(4992, 128)], sem).wait()

        def compute_pair(g, buf_v):
            i0 = g * 2
            i1 = i0 + 1
            slot0 = (i0 % 3) * RINGW
            slot1 = (i1 % 3) * RINGW
            rn0 = (i0 <= 65) & (t0 + i0 <= 127)
            rn1 = (i1 <= 65) & (t0 + i1 <= 127)

            def bind_chunk(j, subs):
                o = j * 16
                k0 = key_v[0, pl.ds(o, 16)]
                k1 = key_v[1, pl.ds(o, 16)]
                k2 = key_v[2, pl.ds(o, 16)]
                k3 = key_v[3, pl.ds(o, 16)]
                for sub, slot in subs:
                    r0 = sub * 4
                    sv = ((buf_v[r0 + 0, pl.ds(o, 16)] * k0
                           + buf_v[r0 + 1, pl.ds(o, 16)] * k1)
                          + (buf_v[r0 + 2, pl.ds(o, 16)] * k2
                             + buf_v[r0 + 3, pl.ds(o, 16)] * k3))
                    ring_v[pl.ds(slot + o, 16)] = sv

            @pl.when(rn1)
            def _():
                @plsc.parallel_loop(0, NCHUNK, unroll=8)
                def bind_both(j):
                    bind_chunk(j, ((0, slot0), (1, slot1)))

            @pl.when(rn0 & jnp.logical_not(rn1))
            def _():
                @plsc.parallel_loop(0, NCHUNK, unroll=8)
                def bind_first(j):
                    bind_chunk(j, ((0, slot0),))

            for i, rn, slot in ((i0, rn0, slot0), (i1, rn1, slot1)):
                @pl.when(rn & (i >= 2))
                def _(i=i, slot=slot):
                    s0 = ((i - 2) % 3) * RINGW
                    s1 = ((i - 1) % 3) * RINGW
                    s2 = slot

                    @plsc.parallel_loop(0, NCHUNK, unroll=8)
                    def win_body(j):
                        o = j * 16
                        a = ring_v[pl.ds(s0 + o, 16)]
                        bb = ring_v[pl.ds(s1 + o + 1, 16)]
                        cc = ring_v[pl.ds(s2 + o + 2, 16)]
                        plsc.addupdate(acc_v.at[pl.ds(o, 16)], a * bb * cc)

        issue(0, buf_a, sem_a)
        issue(1, buf_b, sem_b)

        def pair_body(h, _):
            drain(buf_a, sem_a)
            compute_pair(2 * h, buf_a)

            @pl.when(2 * h + 2 < NP)
            def _():
                issue(2 * h + 2, buf_a, sem_a)

            drain(buf_b, sem_b)
            compute_pair(2 * h + 1, buf_b)

            @pl.when(2 * h + 3 < NP)
            def _():
                issue(2 * h + 3, buf_b, sem_b)

            return 0

        lax.fori_loop(0, NP // 2, pair_body, 0)

        wid = c * 16 + s
        pltpu.sync_copy(acc_v, out_hbm.at[pl.ds(wid * HALF, HALF)])

    return k(e_tbl, halo_tbl, idx_flat, k_flat)


def kernel(signals, feat, keys_w, embed_w, mfcc_w, sin_w, sin_b):
    xs = jnp.clip(signals, 0.0, 1.0)
    idx = jnp.round(xs * (LEVELS - 1)).astype(jnp.int32)
    idx = jnp.clip(idx, 0, LEVELS - 1).reshape(1024, 4)
    idx = jnp.pad(idx, ((0, 8), (0, 0)))
    idx_flat = idx.reshape(-1)

    halo_tbl = jnp.concatenate([embed_w[:, 9984:], embed_w[:, :112]], axis=1)
    k_flat = jnp.concatenate(
        [keys_w[:, :HALF],
         jnp.concatenate([keys_w[:, 4992:], keys_w[:, :112]], axis=1)], axis=0)
    partials = _sc_partials(embed_w, halo_tbl, idx_flat, k_flat)

    u = partials.reshape(2, 16, HALF).sum(axis=1)
    u = jnp.concatenate([u[0, :5000], u[1, 8:5008]])
    s_hv = jnp.roll(u, 2)

    fh = {}
    for si in (0, 3, 4, 5, 6, 11, 12, 15, 17):
        x = feat[546 + 3 * si: 549 + 3 * si]
        proj = sin_w[si] @ x
        fh[si] = jnp.cos(proj + sin_b[si]) * jnp.sin(proj)
    f = ((fh[0] + fh[15] + fh[17]) * (fh[3] + fh[4] + fh[5])
         * fh[6] * (fh[11] + fh[12]))

    return jnp.where((s_hv != 0) & (f > 0), 1.0, -1.0).astype(jnp.float32)


# speedup vs baseline: 1.3639x; 1.0018x over previous
import functools

import jax
import jax.numpy as jnp
import numpy as np
from jax import lax
from jax.experimental import pallas as pl
from jax.experimental.pallas import tpu as pltpu
from jax.experimental.pallas import tpu_sc as plsc

D = 10000
LEVELS = 1024
HALF = 5120
RINGW = 5136
NT = 68
NCHUNK = HALF // 16
SIGNM = np.int32(-(2**31))


def _sc_partials(e_tbl, halo_tbl, idx_flat, k_flat):
    mesh = plsc.VectorSubcoreMesh(core_axis_name="c", subcore_axis_name="s")

    @functools.partial(
        pl.kernel,
        mesh=mesh,
        out_type=jax.ShapeDtypeStruct((32 * HALF,), jnp.float32),
        scratch_types=[
            pltpu.VMEM((272,), jnp.int32),
            pltpu.VMEM((4, HALF), jnp.float32),
            pltpu.VMEM((8, HALF), jnp.float32),
            pltpu.VMEM((8, HALF), jnp.float32),
            pltpu.VMEM((4 * RINGW,), jnp.float32),
            pltpu.VMEM((HALF,), jnp.float32),
            pltpu.SemaphoreType.DMA,
            pltpu.SemaphoreType.DMA,
        ],
    )
    def k(e_hbm, h_hbm, idx_hbm, k_hbm, out_hbm, idx_v, key_v, buf_a, buf_b, ring_v, acc_v, sem_a, sem_b):
        c = lax.axis_index("c")
        s = lax.axis_index("s")
        b = s // 2
        tc = s % 2
        t0 = tc * 64

        pltpu.sync_copy(idx_hbm.at[pl.ds((b * 128 + t0) * 4, 272)], idx_v)
        pltpu.sync_copy(k_hbm.at[pl.ds(c * 4, 4)], key_v)


        zeros = jnp.zeros((16,), jnp.float32)

        def zero_body(j, _):
            o = j * 16
            acc_v[pl.ds(o, 16)] = zeros
            ring_v[pl.ds(o, 16)] = zeros
            ring_v[pl.ds(RINGW + o, 16)] = zeros
            ring_v[pl.ds(2 * RINGW + o, 16)] = zeros
            ring_v[pl.ds(3 * RINGW + o, 16)] = zeros
            return 0

        lax.fori_loop(0, NCHUNK + 1, zero_body, 0)

        NP = NT // 2

        def issue(p, buf, sem):
            idx_sl = idx_v.at[pl.ds(p * 8, 8)]

            @pl.when(c == 0)
            def _():
                pltpu.async_copy(
                    e_hbm.at[idx_sl, pl.ds(0, HALF)], buf, sem)

            @pl.when(c == 1)
            def _():
                pltpu.async_copy(
                    e_hbm.at[idx_sl, pl.ds(4992, 4992)],
                    buf.at[:, pl.ds(0, 4992)], sem)
                pltpu.async_copy(
                    h_hbm.at[idx_sl], buf.at[:, pl.ds(4992, 128)], sem)

        def drain(buf, sem):
            idx_sl = idx_v.at[pl.ds(0, 8)]

            @pl.when(c == 0)
            def _():
                pltpu.make_async_copy(
                    e_hbm.at[idx_sl, pl.ds(0, HALF)], buf, sem).wait()

            @pl.when(c == 1)
            def _():
                pltpu.make_async_copy(
                    e_hbm.at[idx_sl, pl.ds(4992, 4992)],
                    buf.at[:, pl.ds(0, 4992)], sem).wait()
                pltpu.make_async_copy(
                    h_hbm.at[idx_sl], buf.at[:, pl.ds(4992, 128)], sem).wait()

        def compute_pair(g, buf_v):
            i0 = g * 2
            i1 = i0 + 1
            slot0 = (i0 % 4) * RINGW
            slot1 = (i1 % 4) * RINGW
            rn0 = (i0 <= 65) & (t0 + i0 <= 127)
            rn1 = (i1 <= 65) & (t0 + i1 <= 127)

            def bind_chunk(j, subs):
                o = j * 16
                k0 = key_v[0, pl.ds(o, 16)]
                k1 = key_v[1, pl.ds(o, 16)]
                k2 = key_v[2, pl.ds(o, 16)]
                k3 = key_v[3, pl.ds(o, 16)]
                for sub, slot in subs:
                    r0 = sub * 4
                    sv = ((buf_v[r0 + 0, pl.ds(o, 16)] * k0
                           + buf_v[r0 + 1, pl.ds(o, 16)] * k1)
                          + (buf_v[r0 + 2, pl.ds(o, 16)] * k2
                             + buf_v[r0 + 3, pl.ds(o, 16)] * k3))
                    ring_v[pl.ds(slot + o, 16)] = sv

            @pl.when(rn1)
            def _():
                @plsc.parallel_loop(0, NCHUNK, unroll=8)
                def bind_both(j):
                    bind_chunk(j, ((0, slot0), (1, slot1)))

            @pl.when(rn0 & jnp.logical_not(rn1))
            def _():
                @plsc.parallel_loop(0, NCHUNK, unroll=8)
                def bind_first(j):
                    bind_chunk(j, ((0, slot0),))

            for i, rn, slot in ((i0, rn0, slot0), (i1, rn1, slot1)):
                @pl.when(rn & (i >= 2))
                def _(i=i, slot=slot):
                    s0 = ((i - 2) % 4) * RINGW
                    s1 = ((i - 1) % 4) * RINGW
                    s2 = slot

                    @plsc.parallel_loop(0, NCHUNK, unroll=8)
                    def win_body(j):
                        o = j * 16
                        a = ring_v[pl.ds(s0 + o, 16)]
                        bb = ring_v[pl.ds(s1 + o + 1, 16)]
                        cc = ring_v[pl.ds(s2 + o + 2, 16)]
                        plsc.addupdate(acc_v.at[pl.ds(o, 16)], a * bb * cc)

        issue(0, buf_a, sem_a)
        issue(1, buf_b, sem_b)

        def pair_body(h, _):
            drain(buf_a, sem_a)
            compute_pair(2 * h, buf_a)

            @pl.when(2 * h + 2 < NP)
            def _():
                issue(2 * h + 2, buf_a, sem_a)

            drain(buf_b, sem_b)
            compute_pair(2 * h + 1, buf_b)

            @pl.when(2 * h + 3 < NP)
            def _():
                issue(2 * h + 3, buf_b, sem_b)

            return 0

        lax.fori_loop(0, NP // 2, pair_body, 0)

        wid = c * 16 + s
        pltpu.sync_copy(acc_v, out_hbm.at[pl.ds(wid * HALF, HALF)])

    return k(e_tbl, halo_tbl, idx_flat, k_flat)


def kernel(signals, feat, keys_w, embed_w, mfcc_w, sin_w, sin_b):
    xs = jnp.clip(signals, 0.0, 1.0)
    idx = jnp.round(xs * (LEVELS - 1)).astype(jnp.int32)
    idx = jnp.clip(idx, 0, LEVELS - 1).reshape(1024, 4)
    idx = jnp.pad(idx, ((0, 8), (0, 0)))
    idx_flat = idx.reshape(-1)

    halo_tbl = jnp.concatenate([embed_w[:, 9984:], embed_w[:, :112]], axis=1)
    k_flat = jnp.concatenate(
        [keys_w[:, :HALF],
         jnp.concatenate([keys_w[:, 4992:], keys_w[:, :112]], axis=1)], axis=0)
    partials = _sc_partials(embed_w, halo_tbl, idx_flat, k_flat)

    u = partials.reshape(2, 16, HALF).sum(axis=1)
    u = jnp.concatenate([u[0, :5000], u[1, 8:5008]])
    s_hv = jnp.roll(u, 2)

    fh = {}
    for si in (0, 3, 4, 5, 6, 11, 12, 15, 17):
        x = feat[546 + 3 * si: 549 + 3 * si]
        proj = sin_w[si] @ x
        fh[si] = jnp.cos(proj + sin_b[si]) * jnp.sin(proj)
    f = ((fh[0] + fh[15] + fh[17]) * (fh[3] + fh[4] + fh[5])
         * fh[6] * (fh[11] + fh[12]))

    return jnp.where((s_hv != 0) & (f > 0), 1.0, -1.0).astype(jnp.float32)


# speedup vs baseline: 1.3658x; 1.0014x over previous
import functools

import jax
import jax.numpy as jnp
from jax import lax
from jax.experimental import pallas as pl
from jax.experimental.pallas import tpu as pltpu
from jax.experimental.pallas import tpu_sc as plsc

D = 10000
LEVELS = 1024
HALF = 5120
RINGW = 5136
NT = 68
NCHUNK = HALF // 16


def _sc_partials(e_tbl, halo_tbl, idx_flat, k_flat):
    mesh = plsc.VectorSubcoreMesh(core_axis_name="c", subcore_axis_name="s")

    @functools.partial(
        pl.kernel,
        mesh=mesh,
        out_type=jax.ShapeDtypeStruct((32 * HALF,), jnp.float32),
        scratch_types=[
            pltpu.VMEM((272,), jnp.int32),
            pltpu.VMEM((4, HALF), jnp.float32),
            pltpu.VMEM((8, HALF), jnp.float32),
            pltpu.VMEM((8, HALF), jnp.float32),
            pltpu.VMEM((4 * RINGW,), jnp.float32),
            pltpu.VMEM((HALF,), jnp.float32),
            pltpu.SemaphoreType.DMA,
            pltpu.SemaphoreType.DMA,
        ],
    )
    def k(e_hbm, h_hbm, idx_hbm, k_hbm, out_hbm, idx_v, key_v, buf_a, buf_b, ring_v, acc_v, sem_a, sem_b):
        c = lax.axis_index("c")
        s = lax.axis_index("s")
        b = s // 2
        tc = s % 2
        t0 = tc * 64

        pltpu.sync_copy(idx_hbm.at[pl.ds((b * 128 + t0) * 4, 272)], idx_v)
        pltpu.sync_copy(k_hbm.at[pl.ds(c * 4, 4)], key_v)


        zeros = jnp.zeros((16,), jnp.float32)

        def zero_body(j, _):
            o = j * 16
            acc_v[pl.ds(o, 16)] = zeros
            ring_v[pl.ds(o, 16)] = zeros
            ring_v[pl.ds(RINGW + o, 16)] = zeros
            ring_v[pl.ds(2 * RINGW + o, 16)] = zeros
            ring_v[pl.ds(3 * RINGW + o, 16)] = zeros
            return 0

        lax.fori_loop(0, NCHUNK + 1, zero_body, 0)

        NP = NT // 2

        def issue(p, buf, sem):
            idx_sl = idx_v.at[pl.ds(p * 8, 8)]

            @pl.when(c == 0)
            def _():
                pltpu.async_copy(
                    e_hbm.at[idx_sl, pl.ds(0, HALF)], buf, sem)

            @pl.when(c == 1)
            def _():
                pltpu.async_copy(
                    e_hbm.at[idx_sl, pl.ds(4992, 4992)],
                    buf.at[:, pl.ds(0, 4992)], sem)
                pltpu.async_copy(
                    h_hbm.at[idx_sl], buf.at[:, pl.ds(4992, 128)], sem)

        def drain(buf, sem):
            idx_sl = idx_v.at[pl.ds(0, 8)]

            @pl.when(c == 0)
            def _():
                pltpu.make_async_copy(
                    e_hbm.at[idx_sl, pl.ds(0, HALF)], buf, sem).wait()

            @pl.when(c == 1)
            def _():
                pltpu.make_async_copy(
                    e_hbm.at[idx_sl, pl.ds(4992, 4992)],
                    buf.at[:, pl.ds(0, 4992)], sem).wait()
                pltpu.make_async_copy(
                    h_hbm.at[idx_sl], buf.at[:, pl.ds(4992, 128)], sem).wait()

        def compute_pair(g, buf_v):
            i0 = g * 2
            i1 = i0 + 1
            slot0 = (i0 % 4) * RINGW
            slot1 = (i1 % 4) * RINGW
            rn0 = (i0 <= 65) & (t0 + i0 <= 127)
            rn1 = (i1 <= 65) & (t0 + i1 <= 127)

            def bind_chunk(j, subs):
                o = j * 16
                k0 = key_v[0, pl.ds(o, 16)]
                k1 = key_v[1, pl.ds(o, 16)]
                k2 = key_v[2, pl.ds(o, 16)]
                k3 = key_v[3, pl.ds(o, 16)]
                for sub, slot in subs:
                    r0 = sub * 4
                    sv = ((buf_v[r0 + 0, pl.ds(o, 16)] * k0
                           + buf_v[r0 + 1, pl.ds(o, 16)] * k1)
                          + (buf_v[r0 + 2, pl.ds(o, 16)] * k2
                             + buf_v[r0 + 3, pl.ds(o, 16)] * k3))
                    ring_v[pl.ds(slot + o, 16)] = sv

            @pl.when(rn1)
            def _():
                @plsc.parallel_loop(0, NCHUNK, unroll=8)
                def bind_both(j):
                    bind_chunk(j, ((0, slot0), (1, slot1)))

            @pl.when(rn0 & jnp.logical_not(rn1))
            def _():
                @plsc.parallel_loop(0, NCHUNK, unroll=8)
                def bind_first(j):
                    bind_chunk(j, ((0, slot0),))

            for i, rn, slot in ((i0, rn0, slot0), (i1, rn1, slot1)):
                @pl.when(rn & (i >= 2))
                def _(i=i, slot=slot):
                    s0 = ((i - 2) % 4) * RINGW
                    s1 = ((i - 1) % 4) * RINGW
                    s2 = slot

                    @plsc.parallel_loop(0, NCHUNK, unroll=8)
                    def win_body(j):
                        o = j * 16
                        a = ring_v[pl.ds(s0 + o, 16)]
                        bb = ring_v[pl.ds(s1 + o + 1, 16)]
                        cc = ring_v[pl.ds(s2 + o + 2, 16)]
                        plsc.addupdate(acc_v.at[pl.ds(o, 16)], a * bb * cc)

        issue(0, buf_a, sem_a)
        issue(1, buf_b, sem_b)

        def pair_body(h, _):
            drain(buf_a, sem_a)
            compute_pair(2 * h, buf_a)

            @pl.when(2 * h + 2 < NP)
            def _():
                issue(2 * h + 2, buf_a, sem_a)

            drain(buf_b, sem_b)
            compute_pair(2 * h + 1, buf_b)

            @pl.when(2 * h + 3 < NP)
            def _():
                issue(2 * h + 3, buf_b, sem_b)

            return 0

        lax.fori_loop(0, NP // 2, pair_body, 0)

        wid = c * 16 + s
        pltpu.sync_copy(acc_v, out_hbm.at[pl.ds(wid * HALF, HALF)])

    return k(e_tbl, halo_tbl, idx_flat, k_flat)


def kernel(signals, feat, keys_w, embed_w, mfcc_w, sin_w, sin_b):
    xs = jnp.clip(signals, 0.0, 1.0)
    idx = jnp.round(xs * (LEVELS - 1)).astype(jnp.int32)
    idx = jnp.clip(idx, 0, LEVELS - 1).reshape(1024, 4)
    idx = jnp.pad(idx, ((0, 8), (0, 0)))
    idx_flat = idx.reshape(-1)

    halo_tbl = jnp.concatenate([embed_w[:, 9984:], embed_w[:, :112]], axis=1)
    k_flat = jnp.concatenate(
        [keys_w[:, :HALF],
         jnp.concatenate([keys_w[:, 4992:], keys_w[:, :112]], axis=1)], axis=0)
    partials = _sc_partials(embed_w, halo_tbl, idx_flat, k_flat)

    u = partials.reshape(2, 16, HALF).sum(axis=1)
    u = jnp.concatenate([u[0, :5000], u[1, 8:5008]])
    s_hv = jnp.roll(u, 2)

    fh = {}
    for si in (0, 3, 4, 5, 6, 11, 12, 15, 17):
        x = feat[546 + 3 * si: 549 + 3 * si]
        proj = sin_w[si] @ x
        fh[si] = jnp.cos(proj + sin_b[si]) * jnp.sin(proj)
    f = ((fh[0] + fh[15] + fh[17]) * (fh[3] + fh[4] + fh[5])
         * fh[6] * (fh[11] + fh[12]))

    return jnp.where((s_hv != 0) & (f > 0), 1.0, -1.0).astype(jnp.float32)
